# Initial kernel scaffold; baseline (speedup 1.0000x reference)
import jax
import jax.numpy as jnp
from jax.experimental import pallas as pl


def kernel(raw_messages, source_nodes, memory_table, neighbor_table, W1_0, b1_0, W2_0, b2_0, msg_W, msg_b, agg_W, agg_b):
    raise NotImplementedError("write your pallas kernel here")


# speedup vs baseline: 5.6932x; 5.6932x over previous
import functools

import jax
import jax.numpy as jnp
from jax import lax
from jax.experimental import pallas as pl
from jax.experimental.pallas import tpu as pltpu
from jax.experimental.pallas import tpu_sc as plsc

B = 16384
K = 20
D = 128
RAW = 256
N_NODES = 100000
NBR_ROWS = (N_NODES * K) // 128
NW = 32
BPW = B // NW
CHUNK = 128
NCHUNK = BPW // CHUNK
NT = BPW // 16
R1 = 4
G = 4
NG = BPW // G
R3 = 4
OUT_ROWS = 128


def _sc_gather_kernel(src_hbm, mem_hbm, nbrflat_hbm, s_out, n_out,
                      idx_v, rows_v, nid_bufs, neigh_v, node_a, node_b,
                      mem_bufs, out_buf, nsem, rsem_a, rsem_b, msem):
    wid = lax.axis_index("s") * 2 + lax.axis_index("c")
    base = wid * BPW
    lanes = lax.iota(jnp.int32, 16)

    pltpu.sync_copy(src_hbm.at[pl.ds(base, BPW)], idx_v)

    def nid_prep(t, s):
        vvec = idx_v[pl.ds(pl.multiple_of(t * 16, 16), 16)]
        f = vvec * K
        r0 = lax.shift_right_logical(f, 7)
        r1 = jnp.minimum(r0 + 1, NBR_ROWS - 1)
        rows_v[s, pl.ds(0, 16)] = r0
        rows_v[s, pl.ds(16, 16)] = r1
        pltpu.async_copy(nbrflat_hbm.at[rows_v.at[s]], nid_bufs.at[s],
                         nsem.at[s])

    def nid_extract(t, s):
        pltpu.make_async_copy(nbrflat_hbm.at[rows_v.at[s]], nid_bufs.at[s],
                              nsem.at[s]).wait()
        vvec = idx_v[pl.ds(pl.multiple_of(t * 16, 16), 16)]
        p = lax.bitwise_and(vvec * K, 127)
        nvec = t * 16 + lanes
        buf = nid_bufs.at[s]
        for j in range(K):
            pj = p + j
            over = pj >= 128
            rowsel = jnp.where(over, lanes + 16, lanes)
            colsel = lax.bitwise_and(pj, 127)
            vals = plsc.load_gather(buf, [rowsel, colsel])
            plsc.store_scatter(neigh_v, [nvec * K + j], vals)

    for t in range(R1):
        nid_prep(t, t)

    def p1_body(t, _):
        s = lax.rem(t, R1)
        nid_extract(t, s)

        @pl.when(t < NT - R1)
        def _():
            nid_prep(t + R1, s)
        return 0

    lax.fori_loop(0, NT, p1_body, 0)

    node_bufs = (node_a, node_b)
    rsems = (rsem_a, rsem_b)

    def node_fire(c):
        pltpu.async_copy(mem_hbm.at[idx_v.at[pl.ds(c * CHUNK, CHUNK)]],
                         node_bufs[c % 2], rsems[c % 2])

    node_fire(0)
    for c in range(NCHUNK):
        if c + 1 < NCHUNK:
            node_fire(c + 1)
        pltpu.make_async_copy(mem_hbm.at[idx_v.at[pl.ds(c * CHUNK, CHUNK)]],
                              node_bufs[c % 2], rsems[c % 2]).wait()
        pltpu.sync_copy(node_bufs[c % 2], s_out.at[pl.ds(base + c * CHUNK, CHUNK)])

    def mem_fire(g, s):
        off = pl.multiple_of(g * (G * K), 8)
        pltpu.async_copy(mem_hbm.at[neigh_v.at[pl.ds(off, G * K)]],
                         mem_bufs.at[s], msem.at[s])

    for g in range(R3):
        mem_fire(g, g)

    def p3_body(g, _):
        s = lax.rem(g, R3)
        off = pl.multiple_of(g * (G * K), 8)
        pltpu.make_async_copy(mem_hbm.at[neigh_v.at[pl.ds(off, G * K)]],
                              mem_bufs.at[s], msem.at[s]).wait()
        buf = mem_bufs.at[s]
        for n in range(G):
            row = lax.rem(g * G + n, OUT_ROWS)
            for d in range(D // 16):
                sl = pl.ds(d * 16, 16)
                acc = buf[n * K, sl]
                for k in range(1, K):
                    acc = acc + buf[n * K + k, sl]
                out_buf[row, sl] = acc

        @pl.when(g < NG - R3)
        def _():
            mem_fire(g + R3, s)

        gpf = OUT_ROWS // G
        for h in range(NG // gpf):
            @pl.when(g == h * gpf + gpf - 1)
            def _():
                pltpu.sync_copy(out_buf,
                                n_out.at[pl.ds(base + h * OUT_ROWS, OUT_ROWS)])
        return 0

    lax.fori_loop(0, NG, p3_body, 0)


def _sc_gather(source_nodes, memory_table, neighbor_flat):
    mesh = plsc.VectorSubcoreMesh(core_axis_name="c", subcore_axis_name="s")
    f = pl.kernel(
        _sc_gather_kernel,
        mesh=mesh,
        compiler_params=pltpu.CompilerParams(needs_layout_passes=False),
        out_type=[jax.ShapeDtypeStruct((B, D), jnp.float32),
                  jax.ShapeDtypeStruct((B, D), jnp.float32)],
        scratch_types=[
            pltpu.VMEM((BPW,), jnp.int32),
            pltpu.VMEM((R1, 32), jnp.int32),
            pltpu.VMEM((R1, 32, 128), jnp.int32),
            pltpu.VMEM((BPW * K,), jnp.int32),
            pltpu.VMEM((CHUNK, D), jnp.float32),
            pltpu.VMEM((CHUNK, D), jnp.float32),
            pltpu.VMEM((R3, G * K, D), jnp.float32),
            pltpu.VMEM((OUT_ROWS, D), jnp.float32),
            pltpu.SemaphoreType.DMA((R1,)),
            pltpu.SemaphoreType.DMA,
            pltpu.SemaphoreType.DMA,
            pltpu.SemaphoreType.DMA((R3,)),
        ],
    )
    return f(source_nodes, memory_table, neighbor_flat)


def _tc_dense_kernel(raw_ref, s_ref, n_ref, msgW_ref, msgb_ref, W1_ref,
                     b12_ref, W2_ref, aggW_ref, aggb_ref, out_ref):
    hi = jax.lax.Precision.HIGHEST
    msg = jnp.dot(raw_ref[...], msgW_ref[...], precision=hi,
                  preferred_element_type=jnp.float32) + msgb_ref[...]
    h = (jnp.dot(s_ref[...], W1_ref[...], precision=hi,
                 preferred_element_type=jnp.float32)
         + jnp.dot(n_ref[...], W2_ref[...], precision=hi,
                   preferred_element_type=jnp.float32)
         + b12_ref[...])
    h = jnp.maximum(h, 0.0)
    agg = jnp.dot(h, aggW_ref[...], precision=hi,
                  preferred_element_type=jnp.float32) + aggb_ref[...]
    out_ref[...] = jnp.maximum(msg + agg, 0.0)


def _tc_dense(raw, s_rep, n_rep, msg_W, msg_b, W1, b12, W2, agg_W, agg_b):
    blk = 2048
    grid = (B // blk,)
    return pl.pallas_call(
        _tc_dense_kernel,
        grid=grid,
        in_specs=[
            pl.BlockSpec((blk, RAW), lambda i: (i, 0)),
            pl.BlockSpec((blk, D), lambda i: (i, 0)),
            pl.BlockSpec((blk, D), lambda i: (i, 0)),
            pl.BlockSpec((RAW, D), lambda i: (0, 0)),
            pl.BlockSpec((1, D), lambda i: (0, 0)),
            pl.BlockSpec((D, D), lambda i: (0, 0)),
            pl.BlockSpec((1, D), lambda i: (0, 0)),
            pl.BlockSpec((D, D), lambda i: (0, 0)),
            pl.BlockSpec((D, D), lambda i: (0, 0)),
            pl.BlockSpec((1, D), lambda i: (0, 0)),
        ],
        out_specs=pl.BlockSpec((blk, D), lambda i: (i, 0)),
        out_shape=jax.ShapeDtypeStruct((B, D), jnp.float32),
    )(raw, s_rep, n_rep, msg_W, msg_b, W1, b12, W2, agg_W, agg_b)


def kernel(raw_messages, source_nodes, memory_table, neighbor_table,
           W1_0, b1_0, W2_0, b2_0, msg_W, msg_b, agg_W, agg_b):
    neighbor_flat = neighbor_table.reshape(NBR_ROWS, 128)
    s_rep, n_rep = _sc_gather(source_nodes, memory_table, neighbor_flat)
    b12 = (b1_0 + b2_0).reshape(1, D)
    return _tc_dense(raw_messages, s_rep, n_rep, msg_W, msg_b.reshape(1, D),
                     W1_0, b12, W2_0, agg_W, agg_b.reshape(1, D))


# speedup vs baseline: 6.2253x; 1.0935x over previous
import functools

import jax
import jax.numpy as jnp
from jax import lax
from jax.experimental import pallas as pl
from jax.experimental.pallas import tpu as pltpu
from jax.experimental.pallas import tpu_sc as plsc

B = 16384
K = 20
D = 128
RAW = 256
N_NODES = 100000
NBR_ROWS = (N_NODES * K) // 128
NW = 32
BPW = B // NW
CHUNK = 128
NCHUNK = BPW // CHUNK
NT = BPW // 16
R1 = 2
G = 4
NG = BPW // G
R3 = 6
OUT_ROWS = 128


def _sc_gather_kernel(src_hbm, mem_hbm, nbrflat_hbm, s_out, n_out,
                      idx_v, rows_v, nid_bufs, neigh_v, node_a, node_b,
                      mem_bufs, out_buf, nsem, rsem_a, rsem_b, msem):
    wid = lax.axis_index("s") * 2 + lax.axis_index("c")
    base = wid * BPW
    lanes = lax.iota(jnp.int32, 16)

    pltpu.sync_copy(src_hbm.at[pl.ds(base, BPW)], idx_v)

    node_bufs = (node_a, node_b)
    rsems = (rsem_a, rsem_b)

    def node_fire(c):
        pltpu.async_copy(mem_hbm.at[idx_v.at[pl.ds(c * CHUNK, CHUNK)]],
                         node_bufs[c % 2], rsems[c % 2])

    node_fire(0)
    node_fire(1)

    def nid_prep(t, s):
        vvec = idx_v[pl.ds(pl.multiple_of(t * 16, 16), 16)]
        f = vvec * K
        r0 = lax.shift_right_logical(f, 7)
        r1 = jnp.minimum(r0 + 1, NBR_ROWS - 1)
        rows_v[s, pl.ds(0, 16)] = r0
        rows_v[s, pl.ds(16, 16)] = r1
        pltpu.async_copy(nbrflat_hbm.at[rows_v.at[s]], nid_bufs.at[s],
                         nsem.at[s])

    def nid_extract(t, s):
        pltpu.make_async_copy(nbrflat_hbm.at[rows_v.at[s]], nid_bufs.at[s],
                              nsem.at[s]).wait()
        vvec = idx_v[pl.ds(pl.multiple_of(t * 16, 16), 16)]
        p = lax.bitwise_and(vvec * K, 127)
        nvec = t * 16 + lanes
        buf = nid_bufs.at[s]
        for j in range(K):
            pj = p + j
            over = pj >= 128
            rowsel = jnp.where(over, lanes + 16, lanes)
            colsel = lax.bitwise_and(pj, 127)
            vals = plsc.load_gather(buf, [rowsel, colsel])
            plsc.store_scatter(neigh_v, [nvec * K + j], vals)

    for t in range(R1):
        nid_prep(t, t)

    def p1_body(t, _):
        s = lax.rem(t, R1)
        nid_extract(t, s)

        @pl.when(t < NT - R1)
        def _():
            nid_prep(t + R1, s)
        return 0

    lax.fori_loop(0, NT, p1_body, 0)

    for c in range(NCHUNK):
        if c + 2 < NCHUNK:
            node_fire(c + 2)
        pltpu.make_async_copy(mem_hbm.at[idx_v.at[pl.ds(c * CHUNK, CHUNK)]],
                              node_bufs[c % 2], rsems[c % 2]).wait()
        pltpu.sync_copy(node_bufs[c % 2], s_out.at[pl.ds(base + c * CHUNK, CHUNK)])

    def mem_fire(g, s):
        off = pl.multiple_of(g * (G * K), 8)
        pltpu.async_copy(mem_hbm.at[neigh_v.at[pl.ds(off, G * K)]],
                         mem_bufs.at[s], msem.at[s])

    for g in range(R3):
        mem_fire(g, g)

    def p3_body(g, _):
        s = lax.rem(g, R3)
        off = pl.multiple_of(g * (G * K), 8)
        pltpu.make_async_copy(mem_hbm.at[neigh_v.at[pl.ds(off, G * K)]],
                              mem_bufs.at[s], msem.at[s]).wait()
        buf = mem_bufs.at[s]
        for n in range(G):
            row = lax.rem(g * G + n, OUT_ROWS)
            for d in range(D // 16):
                sl = pl.ds(d * 16, 16)
                acc = buf[n * K, sl]
                for k in range(1, K):
                    acc = acc + buf[n * K + k, sl]
                out_buf[row, sl] = acc

        @pl.when(g < NG - R3)
        def _():
            mem_fire(g + R3, s)

        gpf = OUT_ROWS // G
        for h in range(NG // gpf):
            @pl.when(g == h * gpf + gpf - 1)
            def _():
                pltpu.sync_copy(out_buf,
                                n_out.at[pl.ds(base + h * OUT_ROWS, OUT_ROWS)])
        return 0

    lax.fori_loop(0, NG, p3_body, 0)


def _sc_gather(source_nodes, memory_table, neighbor_flat):
    mesh = plsc.VectorSubcoreMesh(core_axis_name="c", subcore_axis_name="s")
    f = pl.kernel(
        _sc_gather_kernel,
        mesh=mesh,
        compiler_params=pltpu.CompilerParams(needs_layout_passes=False),
        out_type=[jax.ShapeDtypeStruct((B, D), jnp.float32),
                  jax.ShapeDtypeStruct((B, D), jnp.float32)],
        scratch_types=[
            pltpu.VMEM((BPW,), jnp.int32),
            pltpu.VMEM((R1, 32), jnp.int32),
            pltpu.VMEM((R1, 32, 128), jnp.int32),
            pltpu.VMEM((BPW * K,), jnp.int32),
            pltpu.VMEM((CHUNK, D), jnp.float32),
            pltpu.VMEM((CHUNK, D), jnp.float32),
            pltpu.VMEM((R3, G * K, D), jnp.float32),
            pltpu.VMEM((OUT_ROWS, D), jnp.float32),
            pltpu.SemaphoreType.DMA((R1,)),
            pltpu.SemaphoreType.DMA,
            pltpu.SemaphoreType.DMA,
            pltpu.SemaphoreType.DMA((R3,)),
        ],
    )
    return f(source_nodes, memory_table, neighbor_flat)


def _tc_dense_kernel(raw_ref, s_ref, n_ref, msgW_ref, msgb_ref, W1_ref,
                     b12_ref, W2_ref, aggW_ref, aggb_ref, out_ref):
    msg = jnp.dot(raw_ref[...], msgW_ref[...],
                  preferred_element_type=jnp.float32) + msgb_ref[...]
    h = (jnp.dot(s_ref[...], W1_ref[...],
                 preferred_element_type=jnp.float32)
         + jnp.dot(n_ref[...], W2_ref[...],
                   preferred_element_type=jnp.float32)
         + b12_ref[...])
    h = jnp.maximum(h, 0.0)
    agg = jnp.dot(h, aggW_ref[...],
                  preferred_element_type=jnp.float32) + aggb_ref[...]
    out_ref[...] = jnp.maximum(msg + agg, 0.0)


def _tc_dense(raw, s_rep, n_rep, msg_W, msg_b, W1, b12, W2, agg_W, agg_b):
    blk = 2048
    grid = (B // blk,)
    return pl.pallas_call(
        _tc_dense_kernel,
        grid=grid,
        in_specs=[
            pl.BlockSpec((blk, RAW), lambda i: (i, 0)),
            pl.BlockSpec((blk, D), lambda i: (i, 0)),
            pl.BlockSpec((blk, D), lambda i: (i, 0)),
            pl.BlockSpec((RAW, D), lambda i: (0, 0)),
            pl.BlockSpec((1, D), lambda i: (0, 0)),
            pl.BlockSpec((D, D), lambda i: (0, 0)),
            pl.BlockSpec((1, D), lambda i: (0, 0)),
            pl.BlockSpec((D, D), lambda i: (0, 0)),
            pl.BlockSpec((D, D), lambda i: (0, 0)),
            pl.BlockSpec((1, D), lambda i: (0, 0)),
        ],
        out_specs=pl.BlockSpec((blk, D), lambda i: (i, 0)),
        out_shape=jax.ShapeDtypeStruct((B, D), jnp.float32),
    )(raw, s_rep, n_rep, msg_W, msg_b, W1, b12, W2, agg_W, agg_b)


def kernel(raw_messages, source_nodes, memory_table, neighbor_table,
           W1_0, b1_0, W2_0, b2_0, msg_W, msg_b, agg_W, agg_b):
    neighbor_flat = neighbor_table.reshape(NBR_ROWS, 128)
    s_rep, n_rep = _sc_gather(source_nodes, memory_table, neighbor_flat)
    b12 = (b1_0 + b2_0).reshape(1, D)
    return _tc_dense(raw_messages, s_rep, n_rep, msg_W, msg_b.reshape(1, D),
                     W1_0, b12, W2_0, agg_W, agg_b.reshape(1, D))
